# baseline (speedup 1.0000x reference)
import functools

import jax
import jax.numpy as jnp
from jax import lax
from jax.experimental import pallas as pl
from jax.experimental.pallas import tpu as pltpu
from jax.experimental.pallas import tpu_sc as plsc

D_MODEL = 1024
SEQ = 4096
BATCH = 16
LANES = 16

NUM_CORES = 2
NUM_SUBCORES = 16
NW = NUM_CORES * NUM_SUBCORES

CHUNK = 64
ZROWS = 32
CHUNKS_PER_TILE = SEQ // (CHUNK * NW)
POS_PER_TILE = SEQ // NW
VPR = D_MODEL // LANES

_MESH = plsc.VectorSubcoreMesh(core_axis_name="c", subcore_axis_name="s")


@functools.partial(
    pl.kernel,
    mesh=_MESH,
    out_type=(
        jax.ShapeDtypeStruct((BATCH, SEQ, D_MODEL), jnp.float32),
        jax.ShapeDtypeStruct((BATCH * SEQ,), jnp.int32),
    ),
    scratch_types=[
        pltpu.VMEM((LANES,), jnp.int32),
        pltpu.VMEM((CHUNK, D_MODEL), jnp.float32),
        pltpu.VMEM((ZROWS, D_MODEL), jnp.float32),
        pltpu.VMEM((POS_PER_TILE,), jnp.int32),
        pltpu.SemaphoreType.DMA,
    ],
)
def _pe_sc(len_hbm, tbl_hbm, zeros_hbm, emb_hbm, pos_hbm,
           len_v, tbl_v, zero_v, pos_v, wsem):
    wid = lax.axis_index("s") * NUM_CORES + lax.axis_index("c")
    pltpu.sync_copy(len_hbm, len_v)
    pltpu.sync_copy(zeros_hbm, zero_v)
    len_vec = len_v[...]

    s0p = wid * POS_PER_TILE
    for b in range(BATCH):
        lb = len_vec[b]

        def pos_i(i, c, lb=lb):
            p = s0p + i * LANES + 1 + lax.iota(jnp.int32, LANES)
            pos_v[pl.ds(i * LANES, LANES)] = jnp.where(p <= lb, p, 0)
            return c

        lax.fori_loop(0, POS_PER_TILE // LANES, pos_i, 0)
        pltpu.sync_copy(pos_v, pos_hbm.at[pl.ds(b * SEQ + s0p, POS_PER_TILE)])

    for c in range(CHUNKS_PER_TILE):
        s0v = (wid * CHUNKS_PER_TILE + c) * CHUNK
        pltpu.sync_copy(tbl_hbm.at[pl.ds(s0v, CHUNK)], tbl_v)

        conds = []
        for b in range(BATCH):
            lb = len_vec[b]
            n = lb - s0v
            full = n >= CHUNK
            empty = n <= 0
            conds.append((full, empty, n))

            @pl.when(full)
            def _(b=b, s0v=s0v):
                pltpu.async_copy(tbl_v, emb_hbm.at[b, pl.ds(s0v, CHUNK)], wsem)

            @pl.when(empty)
            def _(b=b, s0v=s0v):
                pltpu.async_copy(
                    zero_v, emb_hbm.at[b, pl.ds(s0v, ZROWS)], wsem)
                pltpu.async_copy(
                    zero_v, emb_hbm.at[b, pl.ds(s0v + ZROWS, ZROWS)], wsem)

        for b, (full, empty, n) in enumerate(conds):
            @pl.when(full)
            def _(b=b, s0v=s0v):
                pltpu.make_async_copy(
                    tbl_v, emb_hbm.at[b, pl.ds(s0v, CHUNK)], wsem).wait()

            @pl.when(empty)
            def _(b=b, s0v=s0v):
                pltpu.make_async_copy(
                    zero_v, emb_hbm.at[b, pl.ds(s0v, ZROWS)], wsem).wait()
                pltpu.make_async_copy(
                    zero_v, emb_hbm.at[b, pl.ds(s0v + ZROWS, ZROWS)],
                    wsem).wait()

        for b, (full, empty, n) in enumerate(conds):
            @pl.when(jnp.logical_and(n > 0, n < CHUNK))
            def _(b=b, s0v=s0v, n=n):
                def row(r, rc):
                    def col(j, cc):
                        tbl_v[r, pl.ds(j * LANES, LANES)] = jnp.zeros(
                            (LANES,), jnp.float32)
                        return cc

                    lax.fori_loop(0, VPR, col, 0)
                    return rc

                lax.fori_loop(n, CHUNK, row, 0)
                pltpu.sync_copy(tbl_v, emb_hbm.at[b, pl.ds(s0v, CHUNK)])
                pltpu.sync_copy(tbl_hbm.at[pl.ds(s0v, CHUNK)], tbl_v)


def kernel(input_len, table):
    len32 = input_len.astype(jnp.int32)
    tbl = table[1:]
    zeros = jnp.zeros((ZROWS, D_MODEL), jnp.float32)
    emb, pos_flat = _pe_sc(len32, tbl, zeros)
    return emb, pos_flat.reshape(BATCH, SEQ)


# speedup vs baseline: 1.1280x; 1.1280x over previous
import functools

import jax
import jax.numpy as jnp
from jax import lax
from jax.experimental import pallas as pl
from jax.experimental.pallas import tpu as pltpu
from jax.experimental.pallas import tpu_sc as plsc

D_MODEL = 1024
SEQ = 4096
BATCH = 16
LANES = 16

NUM_CORES = 2
NUM_SUBCORES = 16
NW = NUM_CORES * NUM_SUBCORES

CHUNK = 32
CHUNKS_PER_TILE = SEQ // (CHUNK * NW)
POS_PER_TILE = SEQ // NW
VPR = D_MODEL // LANES

_MESH = plsc.VectorSubcoreMesh(core_axis_name="c", subcore_axis_name="s")


@functools.partial(
    pl.kernel,
    mesh=_MESH,
    out_type=(
        jax.ShapeDtypeStruct((BATCH, SEQ, D_MODEL), jnp.float32),
        jax.ShapeDtypeStruct((BATCH * SEQ,), jnp.int32),
    ),
    scratch_types=[
        pltpu.VMEM((LANES,), jnp.int32),
        pltpu.VMEM((CHUNK, D_MODEL), jnp.float32),
        pltpu.VMEM((CHUNK, D_MODEL), jnp.float32),
        pltpu.VMEM((CHUNK, D_MODEL), jnp.float32),
        pltpu.VMEM((POS_PER_TILE,), jnp.int32),
        pltpu.SemaphoreType.DMA,
        pltpu.SemaphoreType.DMA,
        pltpu.SemaphoreType.DMA,
    ],
)
def _pe_sc(len_hbm, tbl_hbm, zeros_hbm, emb_hbm, pos_hbm,
           len_v, tbl_a, tbl_b, zero_v, pos_v, lsem, wsem_a, wsem_b):
    wid = lax.axis_index("s") * NUM_CORES + lax.axis_index("c")
    pltpu.sync_copy(len_hbm, len_v)
    pltpu.sync_copy(zeros_hbm, zero_v)
    len_vec = len_v[...]

    bufs = (tbl_a, tbl_b)
    wsems = (wsem_a, wsem_b)

    def chunk_s0(c):
        return (wid * CHUNKS_PER_TILE + c) * CHUNK

    pltpu.async_copy(tbl_hbm.at[pl.ds(chunk_s0(0), CHUNK)], bufs[0], lsem)

    s0p = wid * POS_PER_TILE
    for b in range(BATCH):
        lb = len_vec[b]

        def pos_i(i, c, lb=lb):
            p = s0p + i * LANES + 1 + lax.iota(jnp.int32, LANES)
            pos_v[pl.ds(i * LANES, LANES)] = jnp.where(p <= lb, p, 0)
            return c

        lax.fori_loop(0, POS_PER_TILE // LANES, pos_i, 0)
        pltpu.sync_copy(pos_v, pos_hbm.at[pl.ds(b * SEQ + s0p, POS_PER_TILE)])

    def drain_writes(c):
        s0v = chunk_s0(c)
        buf = bufs[c % 2]
        wsem = wsems[c % 2]
        for b in range(BATCH):
            lb = len_vec[b]
            n = lb - s0v

            @pl.when(n >= CHUNK)
            def _(b=b, s0v=s0v, buf=buf, wsem=wsem):
                pltpu.make_async_copy(
                    buf, emb_hbm.at[b, pl.ds(s0v, CHUNK)], wsem).wait()

            @pl.when(n <= 0)
            def _(b=b, s0v=s0v, wsem=wsem):
                pltpu.make_async_copy(
                    zero_v, emb_hbm.at[b, pl.ds(s0v, CHUNK)], wsem).wait()

    for c in range(CHUNKS_PER_TILE):
        s0v = chunk_s0(c)
        buf = bufs[c % 2]
        wsem = wsems[c % 2]
        pltpu.make_async_copy(
            tbl_hbm.at[pl.ds(s0v, CHUNK)], buf, lsem).wait()

        for b in range(BATCH):
            lb = len_vec[b]
            n = lb - s0v

            @pl.when(jnp.logical_and(n > 0, n < CHUNK))
            def _(b=b, s0v=s0v, n=n, buf=buf):
                def row(r, rc):
                    def col(j, cc):
                        buf[r, pl.ds(j * LANES, LANES)] = jnp.zeros(
                            (LANES,), jnp.float32)
                        return cc

                    lax.fori_loop(0, VPR, col, 0)
                    return rc

                lax.fori_loop(n, CHUNK, row, 0)
                pltpu.sync_copy(buf, emb_hbm.at[b, pl.ds(s0v, CHUNK)])
                pltpu.sync_copy(tbl_hbm.at[pl.ds(s0v, CHUNK)], buf)

        if c >= 1:
            drain_writes(c - 1)
        if c + 1 < CHUNKS_PER_TILE:
            pltpu.async_copy(
                tbl_hbm.at[pl.ds(chunk_s0(c + 1), CHUNK)],
                bufs[(c + 1) % 2], lsem)

        for b in range(BATCH):
            lb = len_vec[b]
            n = lb - s0v

            @pl.when(n >= CHUNK)
            def _(b=b, s0v=s0v, buf=buf, wsem=wsem):
                pltpu.async_copy(buf, emb_hbm.at[b, pl.ds(s0v, CHUNK)], wsem)

            @pl.when(n <= 0)
            def _(b=b, s0v=s0v, wsem=wsem):
                pltpu.async_copy(
                    zero_v, emb_hbm.at[b, pl.ds(s0v, CHUNK)], wsem)

    drain_writes(CHUNKS_PER_TILE - 1)


def kernel(input_len, table):
    len32 = input_len.astype(jnp.int32)
    tbl = table[1:]
    zeros = jnp.zeros((CHUNK, D_MODEL), jnp.float32)
    emb, pos_flat = _pe_sc(len32, tbl, zeros)
    return emb, pos_flat.reshape(BATCH, SEQ)
